# Initial kernel scaffold; baseline (speedup 1.0000x reference)
import jax
import jax.numpy as jnp
from jax.experimental import pallas as pl


def kernel(x, embedding):
    raise NotImplementedError("write your pallas kernel here")


# speedup vs baseline: 1.3340x; 1.3340x over previous
import functools

import jax
import jax.numpy as jnp
from jax import lax
from jax.experimental import pallas as pl
from jax.experimental.pallas import tpu as pltpu
from jax.experimental.pallas import tpu_sc as plsc

_EN = 8192
_ED = 64
_BM = 256

_NC = 2
_NS = 16
_NW = _NC * _NS
_CHUNK = 128


def _vq_tc_body(z_ref, et_ref, idx_ref, dsum_ref, isum_ref):
    z = z_ref[...]
    et = et_ref[...]
    z2 = jnp.sum(z * z, axis=1, keepdims=True)
    e2 = jnp.sum(et * et, axis=0, keepdims=True)
    mm = lax.dot_general(z, et, (((1,), (0,)), ((), ())),
                         preferred_element_type=jnp.float32)
    dis = (z2 + e2) - 2.0 * mm
    m = jnp.min(dis, axis=1, keepdims=True)
    col = lax.broadcasted_iota(jnp.int32, dis.shape, 1)
    idx = jnp.min(jnp.where(dis == m, col, jnp.int32(2**31 - 1)),
                  axis=1, keepdims=True)
    idx_ref[...] = idx
    dsum_ref[...] = jnp.sum(m, keepdims=True).reshape(1, 1, 1)
    isum_ref[...] = jnp.sum(idx.astype(jnp.float32), keepdims=True).reshape(1, 1, 1)


def _argmin_distances(z, et):
    m_total = z.shape[0]
    grid = (m_total // _BM,)
    return pl.pallas_call(
        _vq_tc_body,
        grid=grid,
        in_specs=[
            pl.BlockSpec((_BM, _ED), lambda i: (i, 0)),
            pl.BlockSpec((_ED, _EN), lambda i: (0, 0)),
        ],
        out_specs=[
            pl.BlockSpec((_BM, 1), lambda i: (i, 0)),
            pl.BlockSpec((1, 1, 1), lambda i: (i, 0, 0)),
            pl.BlockSpec((1, 1, 1), lambda i: (i, 0, 0)),
        ],
        out_shape=[
            jax.ShapeDtypeStruct((m_total, 1), jnp.int32),
            jax.ShapeDtypeStruct((m_total // _BM, 1, 1), jnp.float32),
            jax.ShapeDtypeStruct((m_total // _BM, 1, 1), jnp.float32),
        ],
    )(z, et)


def _make_sc_gather(b_total):
    b_per_w = b_total // _NW
    n_chunks = b_per_w // _CHUNK
    mesh = plsc.VectorSubcoreMesh(core_axis_name="c", subcore_axis_name="s")

    @functools.partial(
        pl.kernel,
        mesh=mesh,
        compiler_params=pltpu.CompilerParams(use_tc_tiling_on_sc=False),
        out_type=jax.ShapeDtypeStruct((b_total, _ED), jnp.float32),
        scratch_types=[
            pltpu.VMEM((n_chunks, _CHUNK), jnp.int32),
            pltpu.VMEM((b_per_w, _ED), jnp.float32),
            pltpu.SemaphoreType.DMA,
        ],
    )
    def gather_k(table_hbm, idx_hbm, out_hbm, idx_v, rows_v, sem):
        wid = lax.axis_index("s") * _NC + lax.axis_index("c")
        pltpu.sync_copy(idx_hbm.at[wid], idx_v)
        copies = []
        for j in range(n_chunks):
            copies.append(pltpu.async_copy(
                table_hbm.at[idx_v.at[j]],
                rows_v.at[pl.ds(j * _CHUNK, _CHUNK)],
                sem))
        for c in copies:
            c.wait()
        pltpu.sync_copy(rows_v, out_hbm.at[pl.ds(wid * b_per_w, b_per_w)])

    return gather_k


def kernel(x, embedding):
    m_total = x.shape[0] * x.shape[1]
    z = x.reshape(m_total, _ED)
    et = embedding.T
    idx2d, dsum, isum = _argmin_distances(z, et)
    idx = idx2d.reshape(m_total)

    idx3 = idx.reshape(_NW, m_total // _NW // _CHUNK, _CHUNK)
    zq = _make_sc_gather(m_total)(embedding, idx3)
    z_q = zq.reshape(x.shape)

    denom = jnp.float32(m_total * _ED)
    loss = (1.0 + 0.25) * jnp.sum(dsum) / denom
    e_min = jnp.sum(isum) / jnp.float32(m_total)
    perplexity = jnp.exp(-(e_min * jnp.log(e_min + 1e-10)))
    return (loss, z_q, perplexity, idx)
